# baseline (speedup 1.0000x reference)
import functools

import jax
import jax.numpy as jnp
from jax import lax
from jax.experimental import pallas as pl
from jax.experimental.pallas import tpu as pltpu
from jax.experimental.pallas import tpu_sc as plsc

N_NODES = 10000
N_EDGES = 320000
D = 128

NC = 2
NS = 16
NW = NC * NS
K = 64
CH_C0 = 162
CH_C1 = 162
CH_MAX = max(CH_C0, CH_C1)
E_PAD = NS * (CH_C0 + CH_C1) * K
N_PAD = 10240
ROWS_PER_TILE = N_PAD // NS


def _matmul_body(x_ref, w_ref, b_ref, out_ref):
    out_ref[...] = (
        lax.dot_general(
            x_ref[...], w_ref[...],
            dimension_numbers=(((1,), (1,)), ((), ())),
            preferred_element_type=jnp.float32,
        )
        + b_ref[...]
    )


def _linear(x_pad, W, b):
    grid = 8
    blk = N_PAD // grid
    spec = pl.BlockSpec((blk, D), lambda i: (i, 0))
    return pl.pallas_call(
        _matmul_body,
        grid=(grid,),
        in_specs=[
            spec,
            pl.BlockSpec((D, D), lambda i: (0, 0)),
            pl.BlockSpec((1, D), lambda i: (0, 0)),
        ],
        out_specs=spec,
        out_shape=jax.ShapeDtypeStruct((N_PAD, D), jnp.float32),
    )(x_pad, W, b.reshape(1, D))


def _sc_scatter_body(xt_hbm, zeros_hbm, src_hbm, dst_hbm, out_hbm,
                     src_v, dst_a, dst_b, dst_c, rows_a, rows_b, rows_c, acc,
                     sem_ga, sem_gb, sem_gc, sem_ia, sem_ib, sem_ic):
    c = lax.axis_index("c")
    s = lax.axis_index("s")
    wid = s * NC + c
    chn = lax.select(c == 0, CH_C0, CH_C1)
    pltpu.sync_copy(src_hbm.at[wid], src_v)
    row0 = s * ROWS_PER_TILE

    @pl.when(c == 0)
    def _():
        pltpu.sync_copy(xt_hbm.at[pl.ds(row0, ROWS_PER_TILE)],
                        acc.at[pl.ds(row0, ROWS_PER_TILE)])

    @pl.when(c != 0)
    def _():
        pltpu.sync_copy(zeros_hbm.at[pl.ds(row0, ROWS_PER_TILE)],
                        acc.at[pl.ds(row0, ROWS_PER_TILE)])
    plsc.subcore_barrier()

    def gather(j, rows, sem):
        return pltpu.async_copy(xt_hbm.at[src_v.at[j]], rows, sem)

    def wait_gather(j, rows, sem):
        pltpu.make_async_copy(xt_hbm.at[src_v.at[j]], rows, sem).wait()

    def load_dst(j, buf, sem):
        return pltpu.async_copy(dst_hbm.at[wid, pl.ds(j, 1)], buf, sem)

    def wait_dst(j, buf, sem):
        pltpu.make_async_copy(dst_hbm.at[wid, pl.ds(j, 1)], buf, sem).wait()

    def scatter(rows, buf):
        pltpu.sync_copy(rows, acc.at[buf.at[0]], add=True)

    load_dst(0, dst_a, sem_ia)
    gather(0, rows_a, sem_ga)
    load_dst(1, dst_b, sem_ib)
    gather(1, rows_b, sem_gb)
    load_dst(2, dst_c, sem_ic)

    def triple(i, carry):
        j = 3 * i
        wait_gather(j, rows_a, sem_ga)
        gather(j + 2, rows_c, sem_gc)
        wait_dst(j, dst_a, sem_ia)
        scatter(rows_a, dst_a)

        @pl.when(j + 3 < chn)
        def _():
            load_dst(j + 3, dst_a, sem_ia)

        wait_gather(j + 1, rows_b, sem_gb)

        @pl.when(j + 3 < chn)
        def _():
            gather(j + 3, rows_a, sem_ga)

        wait_dst(j + 1, dst_b, sem_ib)
        scatter(rows_b, dst_b)

        @pl.when(j + 4 < chn)
        def _():
            load_dst(j + 4, dst_b, sem_ib)

        wait_gather(j + 2, rows_c, sem_gc)

        @pl.when(j + 4 < chn)
        def _():
            gather(j + 4, rows_b, sem_gb)

        wait_dst(j + 2, dst_c, sem_ic)
        scatter(rows_c, dst_c)

        @pl.when(j + 5 < chn)
        def _():
            load_dst(j + 5, dst_c, sem_ic)

        return carry

    lax.fori_loop(0, chn // 3, triple, 0)
    plsc.subcore_barrier()
    pltpu.sync_copy(acc.at[pl.ds(row0, ROWS_PER_TILE)],
                    out_hbm.at[c, pl.ds(row0, ROWS_PER_TILE)])


def _sc_scatter(xt, zeros_hbm, src3, dst3):
    mesh = plsc.VectorSubcoreMesh(
        core_axis_name="c", subcore_axis_name="s",
        num_cores=NC, num_subcores=NS,
    )
    return pl.kernel(
        _sc_scatter_body,
        out_type=jax.ShapeDtypeStruct((NC, N_PAD, D), jnp.float32),
        mesh=mesh,
        scratch_types=[
            pltpu.VMEM((CH_MAX, K), jnp.int32),
            pltpu.VMEM((1, K), jnp.int32),
            pltpu.VMEM((1, K), jnp.int32),
            pltpu.VMEM((1, K), jnp.int32),
            pltpu.VMEM((K, D), jnp.float32),
            pltpu.VMEM((K, D), jnp.float32),
            pltpu.VMEM((K, D), jnp.float32),
            pltpu.VMEM_SHARED((N_PAD, D), jnp.float32),
            pltpu.SemaphoreType.DMA,
            pltpu.SemaphoreType.DMA,
            pltpu.SemaphoreType.DMA,
            pltpu.SemaphoreType.DMA,
            pltpu.SemaphoreType.DMA,
            pltpu.SemaphoreType.DMA,
        ],
    )(xt, zeros_hbm, src3, dst3)


def _combine_body(p0_ref, p1_ref, out_ref):
    out_ref[...] = p0_ref[...] + p1_ref[...]


def _combine(p0, p1):
    grid = 5
    blk = N_NODES // grid
    spec = pl.BlockSpec((blk, D), lambda i: (i, 0))
    return pl.pallas_call(
        _combine_body,
        grid=(grid,),
        in_specs=[spec, spec],
        out_specs=spec,
        out_shape=jax.ShapeDtypeStruct((N_NODES, D), jnp.float32),
    )(p0, p1)


@jax.jit
def kernel(x, edge_index, W, b):
    x_pad = jnp.pad(x, ((0, N_PAD - N_NODES), (0, 0)))
    ei = edge_index.astype(jnp.int32)
    pad = E_PAD - N_EDGES
    dummy_dst = N_NODES + jnp.arange(pad, dtype=jnp.int32) % (N_PAD - N_NODES)
    srcp = jnp.concatenate([ei[0], jnp.zeros((pad,), jnp.int32)])
    dstp = jnp.concatenate([ei[1], dummy_dst])

    def _layout(flat):
        e0, e1 = CH_C0 * K, CH_C1 * K
        e_max = CH_MAX * K
        c0 = flat[:NS * e0].reshape(NS, e0)
        c1 = flat[NS * e0:].reshape(NS, e1)
        c0 = jnp.pad(c0, ((0, 0), (0, e_max - e0)))
        c1 = jnp.pad(c1, ((0, 0), (0, e_max - e1)))
        return jnp.stack([c0, c1], axis=1).reshape(NW, CH_MAX, K)

    src3 = _layout(srcp)
    dst3 = _layout(dstp)
    zeros_hbm = jnp.zeros((N_PAD, D), jnp.float32)

    xt = _linear(x_pad, W, b)
    partials = _sc_scatter(xt, zeros_hbm, src3, dst3)
    return _combine(partials[0, :N_NODES], partials[1, :N_NODES])


# speedup vs baseline: 2.5353x; 2.5353x over previous
import functools

import jax
import jax.numpy as jnp
from jax import lax
from jax.experimental import pallas as pl
from jax.experimental.pallas import tpu as pltpu
from jax.experimental.pallas import tpu_sc as plsc

N_NODES = 10000
N_EDGES = 320000
D = 128

NC = 2
NS = 16
NW = NC * NS
K = 128
CH_C0 = 79
CH_C1 = 79
CH_MAX = max(CH_C0, CH_C1)
E_PAD = NS * (CH_C0 + CH_C1) * K
N_PAD = 10240
ROWS_PER_TILE = N_PAD // NS


def _matmul_body(x_ref, w_ref, b_ref, out_ref):
    out_ref[...] = (
        lax.dot_general(
            x_ref[...], w_ref[...],
            dimension_numbers=(((1,), (1,)), ((), ())),
            preferred_element_type=jnp.float32,
        )
        + b_ref[...]
    )


def _linear(x_pad, W, b):
    grid = 8
    blk = N_PAD // grid
    spec = pl.BlockSpec((blk, D), lambda i: (i, 0))
    return pl.pallas_call(
        _matmul_body,
        grid=(grid,),
        in_specs=[
            spec,
            pl.BlockSpec((D, D), lambda i: (0, 0)),
            pl.BlockSpec((1, D), lambda i: (0, 0)),
        ],
        out_specs=spec,
        out_shape=jax.ShapeDtypeStruct((N_PAD, D), jnp.float32),
    )(x_pad, W, b.reshape(1, D))


def _sc_scatter_body(xt_hbm, zeros_hbm, src_hbm, dst_hbm, out_hbm,
                     src_v, dst_a, dst_b, rows_a, rows_b, acc,
                     sem_g, sem_ia, sem_ib):
    c = lax.axis_index("c")
    s = lax.axis_index("s")
    wid = s * NC + c
    chn = lax.select(c == 0, CH_C0, CH_C1)
    pltpu.sync_copy(src_hbm.at[wid], src_v)
    row0 = s * ROWS_PER_TILE

    @pl.when(c == 0)
    def _():
        pltpu.sync_copy(xt_hbm.at[pl.ds(row0, ROWS_PER_TILE)],
                        acc.at[pl.ds(row0, ROWS_PER_TILE)])

    @pl.when(c != 0)
    def _():
        pltpu.sync_copy(zeros_hbm.at[pl.ds(row0, ROWS_PER_TILE)],
                        acc.at[pl.ds(row0, ROWS_PER_TILE)])
    plsc.subcore_barrier()

    def gather(j, rows):
        return pltpu.async_copy(xt_hbm.at[src_v.at[j]], rows, sem_g)

    def wait_gather(j, rows):
        pltpu.make_async_copy(xt_hbm.at[src_v.at[j]], rows, sem_g).wait()

    def load_dst(j, buf, sem):
        return pltpu.async_copy(dst_hbm.at[wid, pl.ds(j, 1)], buf, sem)

    def wait_dst(j, buf, sem):
        pltpu.make_async_copy(dst_hbm.at[wid, pl.ds(j, 1)], buf, sem).wait()

    def scatter(rows, buf):
        pltpu.sync_copy(rows, acc.at[buf.at[0]], add=True)

    load_dst(0, dst_a, sem_ia)
    gather(0, rows_a)
    load_dst(1, dst_b, sem_ib)

    def pair(i, carry):
        j0 = 2 * i
        j1 = j0 + 1
        wait_gather(j0, rows_a)
        gather(j1, rows_b)
        wait_dst(j0, dst_a, sem_ia)
        scatter(rows_a, dst_a)
        load_dst(j0 + 2, dst_a, sem_ia)
        wait_gather(j1, rows_b)
        gather(j0 + 2, rows_a)
        wait_dst(j1, dst_b, sem_ib)
        scatter(rows_b, dst_b)

        @pl.when(j1 + 2 < chn)
        def _():
            load_dst(j1 + 2, dst_b, sem_ib)

        return carry

    lax.fori_loop(0, chn // 2, pair, 0)
    wait_gather(chn - 1, rows_a)
    wait_dst(chn - 1, dst_a, sem_ia)
    scatter(rows_a, dst_a)
    plsc.subcore_barrier()
    pltpu.sync_copy(acc.at[pl.ds(row0, ROWS_PER_TILE)],
                    out_hbm.at[c, pl.ds(row0, ROWS_PER_TILE)])


def _sc_scatter(xt, zeros_hbm, src3, dst3):
    mesh = plsc.VectorSubcoreMesh(
        core_axis_name="c", subcore_axis_name="s",
        num_cores=NC, num_subcores=NS,
    )
    return pl.kernel(
        _sc_scatter_body,
        out_type=jax.ShapeDtypeStruct((NC, N_PAD, D), jnp.float32),
        mesh=mesh,
        scratch_types=[
            pltpu.VMEM((CH_MAX, K), jnp.int32),
            pltpu.VMEM((1, K), jnp.int32),
            pltpu.VMEM((1, K), jnp.int32),
            pltpu.VMEM((K, D), jnp.float32),
            pltpu.VMEM((K, D), jnp.float32),
            pltpu.VMEM_SHARED((N_PAD, D), jnp.float32),
            pltpu.SemaphoreType.DMA,
            pltpu.SemaphoreType.DMA,
            pltpu.SemaphoreType.DMA,
        ],
    )(xt, zeros_hbm, src3, dst3)


def _combine_body(p0_ref, p1_ref, out_ref):
    out_ref[...] = p0_ref[...] + p1_ref[...]


def _combine(p0, p1):
    grid = 5
    blk = N_NODES // grid
    spec = pl.BlockSpec((blk, D), lambda i: (i, 0))
    return pl.pallas_call(
        _combine_body,
        grid=(grid,),
        in_specs=[spec, spec],
        out_specs=spec,
        out_shape=jax.ShapeDtypeStruct((N_NODES, D), jnp.float32),
    )(p0, p1)


@jax.jit
def kernel(x, edge_index, W, b):
    x_pad = jnp.pad(x, ((0, N_PAD - N_NODES), (0, 0)))
    ei = edge_index.astype(jnp.int32)
    pad = E_PAD - N_EDGES
    dummy_dst = N_NODES + jnp.arange(pad, dtype=jnp.int32) % (N_PAD - N_NODES)
    srcp = jnp.concatenate([ei[0], jnp.zeros((pad,), jnp.int32)])
    dstp = jnp.concatenate([ei[1], dummy_dst])

    def _layout(flat):
        e0, e1 = CH_C0 * K, CH_C1 * K
        e_max = CH_MAX * K
        c0 = flat[:NS * e0].reshape(NS, e0)
        c1 = flat[NS * e0:].reshape(NS, e1)
        c0 = jnp.pad(c0, ((0, 0), (0, e_max - e0)))
        c1 = jnp.pad(c1, ((0, 0), (0, e_max - e1)))
        return jnp.stack([c0, c1], axis=1).reshape(NW, CH_MAX, K)

    src3 = _layout(srcp)
    dst3 = _layout(dstp)
    zeros_hbm = jnp.zeros((N_PAD, D), jnp.float32)

    xt = _linear(x_pad, W, b)
    partials = _sc_scatter(xt, zeros_hbm, src3, dst3)
    return _combine(partials[0, :N_NODES], partials[1, :N_NODES])
